# Initial kernel scaffold; baseline (speedup 1.0000x reference)
import functools

import jax
import jax.numpy as jnp
from jax import lax
from jax.experimental import pallas as pl
from jax.experimental.pallas import tpu as pltpu
from jax.experimental.pallas import tpu_sc as plsc

TOPK = 6
E = 64
H = 2048
ROWS = 32768

TC_BLOCK = 512

NC = 2
NS = 16
NW = NC * NS
RPW = ROWS // NW
L = 16


def _tc_scores_body(w_ref, x_ref, o_ref):
    lt = lax.dot_general(
        w_ref[...], x_ref[...], (((1,), (1,)), ((), ())),
        preferred_element_type=jnp.float32,
    )
    m = jnp.max(lt, axis=0, keepdims=True)
    ex = jnp.exp(lt - m)
    o_ref[...] = ex / jnp.sum(ex, axis=0, keepdims=True)


def _tc_scores(flat, w):
    n_blocks = ROWS // TC_BLOCK
    return pl.pallas_call(
        _tc_scores_body,
        grid=(n_blocks,),
        in_specs=[
            pl.BlockSpec((E, H), lambda i: (0, 0)),
            pl.BlockSpec((TC_BLOCK, H), lambda i: (i, 0)),
        ],
        out_specs=pl.BlockSpec((E, TC_BLOCK), lambda i: (0, i)),
        out_shape=jax.ShapeDtypeStruct((E, ROWS), jnp.float32),
    )(w, flat)


def _sc_topk_body(scores_hbm, idx_hbm, val_hbm, buf, idxb, valb, sem):
    wid = lax.axis_index("c") * NS + lax.axis_index("s")
    base = wid * RPW
    pltpu.sync_copy(scores_hbm.at[:, pl.ds(base, RPW)], buf)

    lane = lax.iota(jnp.int32, L)

    def group_body(g, carry):
        off = g * L
        s = [jnp.full((L,), -1.0, jnp.float32) for _ in range(TOPK)]
        ix = [jnp.zeros((L,), jnp.int32) for _ in range(TOPK)]
        for e in range(E):
            v = buf[e, pl.ds(off, L)]
            c = v > s[TOPK - 1]
            s[TOPK - 1] = jnp.maximum(s[TOPK - 1], v)
            ix[TOPK - 1] = jnp.where(c, jnp.full((L,), e, jnp.int32), ix[TOPK - 1])
            for j in range(TOPK - 1, 0, -1):
                cj = s[j] > s[j - 1]
                hi = jnp.maximum(s[j - 1], s[j])
                lo = jnp.minimum(s[j - 1], s[j])
                s[j - 1], s[j] = hi, lo
                ihi = jnp.where(cj, ix[j], ix[j - 1])
                ilo = jnp.where(cj, ix[j - 1], ix[j])
                ix[j - 1], ix[j] = ihi, ilo
        rows = off + lane
        for j in range(TOPK):
            col = jnp.full((L,), j, jnp.int32)
            plsc.store_scatter(valb, [rows, col], s[j])
            plsc.store_scatter(idxb, [rows, col], ix[j])
        return carry

    lax.fori_loop(0, RPW // L, group_body, 0)

    pltpu.sync_copy(idxb, idx_hbm.at[pl.ds(base, RPW)])
    pltpu.sync_copy(valb, val_hbm.at[pl.ds(base, RPW)])


@functools.partial(
    pl.kernel,
    mesh=plsc.VectorSubcoreMesh(core_axis_name="c", subcore_axis_name="s"),
    out_type=[
        jax.ShapeDtypeStruct((ROWS, TOPK), jnp.int32),
        jax.ShapeDtypeStruct((ROWS, TOPK), jnp.float32),
    ],
    scratch_types=[
        pltpu.VMEM((E, RPW), jnp.float32),
        pltpu.VMEM((RPW, TOPK), jnp.int32),
        pltpu.VMEM((RPW, TOPK), jnp.float32),
        pltpu.SemaphoreType.DMA,
    ],
)
def _sc_topk(scores_hbm, idx_hbm, val_hbm, buf, idxb, valb, sem):
    _sc_topk_body(scores_hbm, idx_hbm, val_hbm, buf, idxb, valb, sem)


def kernel(hidden_states, W):
    flat = hidden_states.reshape(-1, H)
    scores_t = _tc_scores(flat, W)
    topk_idx, topk_val = _sc_topk(scores_t)
    return (topk_idx, topk_val)


# speedup vs baseline: 1.5818x; 1.5818x over previous
import functools

import jax
import jax.numpy as jnp
from jax import lax
from jax.experimental import pallas as pl
from jax.experimental.pallas import tpu as pltpu
from jax.experimental.pallas import tpu_sc as plsc

TOPK = 6
E = 64
H = 2048
ROWS = 32768

TC_BLOCK = 512

NC = 2
NS = 16
NW = NC * NS
RPW = ROWS // NW
L = 16


def _tc_scores_body(w_ref, x_ref, o_ref):
    lt = lax.dot_general(
        w_ref[...], x_ref[...], (((1,), (1,)), ((), ())),
        preferred_element_type=jnp.float32,
    )
    m = jnp.max(lt, axis=0, keepdims=True)
    ex = jnp.exp(lt - m)
    o_ref[...] = ex / jnp.sum(ex, axis=0, keepdims=True)


def _tc_scores(flat, w):
    n_blocks = ROWS // TC_BLOCK
    return pl.pallas_call(
        _tc_scores_body,
        grid=(n_blocks,),
        in_specs=[
            pl.BlockSpec((E, H), lambda i: (0, 0)),
            pl.BlockSpec((TC_BLOCK, H), lambda i: (i, 0)),
        ],
        out_specs=pl.BlockSpec((E, TC_BLOCK), lambda i: (0, i)),
        out_shape=jax.ShapeDtypeStruct((E, ROWS), jnp.float32),
    )(w, flat)


def _sc_topk_body(scores_hbm, idx_hbm, val_hbm, buf, idxb, valb, sem):
    wid = lax.axis_index("c") * NS + lax.axis_index("s")
    base = wid * RPW
    pltpu.sync_copy(scores_hbm.at[:, pl.ds(base, RPW)], buf)

    lane = lax.iota(jnp.int32, L)

    def group_body(g, carry):
        off = g * L
        s = [jnp.full((L,), -1.0, jnp.float32) for _ in range(TOPK)]
        ix = [jnp.zeros((L,), jnp.int32) for _ in range(TOPK)]
        for e in range(E):
            v = buf[e, pl.ds(off, L)]
            c = v > s[TOPK - 1]
            s[TOPK - 1] = jnp.maximum(s[TOPK - 1], v)
            ix[TOPK - 1] = jnp.where(c, jnp.full((L,), e, jnp.int32), ix[TOPK - 1])
            for j in range(TOPK - 1, 0, -1):
                cj = s[j] > s[j - 1]
                hi = jnp.maximum(s[j - 1], s[j])
                lo = jnp.minimum(s[j - 1], s[j])
                s[j - 1], s[j] = hi, lo
                ihi = jnp.where(cj, ix[j], ix[j - 1])
                ilo = jnp.where(cj, ix[j - 1], ix[j])
                ix[j - 1], ix[j] = ihi, ilo
        for j in range(TOPK):
            valb[j, pl.ds(off, L)] = s[j]
            idxb[j, pl.ds(off, L)] = ix[j]
        return carry

    lax.fori_loop(0, RPW // L, group_body, 0)

    pltpu.sync_copy(idxb, idx_hbm.at[:, pl.ds(base, RPW)])
    pltpu.sync_copy(valb, val_hbm.at[:, pl.ds(base, RPW)])


@functools.partial(
    pl.kernel,
    mesh=plsc.VectorSubcoreMesh(core_axis_name="c", subcore_axis_name="s"),
    out_type=[
        jax.ShapeDtypeStruct((TOPK, ROWS), jnp.int32),
        jax.ShapeDtypeStruct((TOPK, ROWS), jnp.float32),
    ],
    scratch_types=[
        pltpu.VMEM((E, RPW), jnp.float32),
        pltpu.VMEM((TOPK, RPW), jnp.int32),
        pltpu.VMEM((TOPK, RPW), jnp.float32),
        pltpu.SemaphoreType.DMA,
    ],
)
def _sc_topk(scores_hbm, idx_hbm, val_hbm, buf, idxb, valb, sem):
    _sc_topk_body(scores_hbm, idx_hbm, val_hbm, buf, idxb, valb, sem)


def kernel(hidden_states, W):
    flat = hidden_states.reshape(-1, H)
    scores_t = _tc_scores(flat, W)
    topk_idx_t, topk_val_t = _sc_topk(scores_t)
    return (topk_idx_t.T, topk_val_t.T)


# speedup vs baseline: 1.7982x; 1.1369x over previous
import functools

import jax
import jax.numpy as jnp
from jax import lax
from jax.experimental import pallas as pl
from jax.experimental.pallas import tpu as pltpu
from jax.experimental.pallas import tpu_sc as plsc

TOPK = 6
E = 64
H = 2048
ROWS = 32768

NCHUNK = 2
CH_ROWS = ROWS // NCHUNK

TC_BLOCK = 2048

NC = 2
NS = 16
NW = NC * NS
RPW = CH_ROWS // NW
L = 16


def _tc_scores_body(w_ref, x_ref, o_ref):
    lt = lax.dot_general(
        w_ref[...], x_ref[...], (((1,), (1,)), ((), ())),
        preferred_element_type=jnp.float32,
    )
    m = jnp.max(lt, axis=0, keepdims=True)
    ex = jnp.exp(lt - m)
    o_ref[...] = ex / jnp.sum(ex, axis=0, keepdims=True)


def _tc_scores(flat, w, chunk):
    n_blocks = CH_ROWS // TC_BLOCK
    base = chunk * n_blocks
    return pl.pallas_call(
        _tc_scores_body,
        grid=(n_blocks,),
        in_specs=[
            pl.BlockSpec((E, H), lambda i: (0, 0)),
            pl.BlockSpec((TC_BLOCK, H), lambda i: (base + i, 0)),
        ],
        out_specs=pl.BlockSpec((E, TC_BLOCK), lambda i: (0, i)),
        out_shape=jax.ShapeDtypeStruct((E, CH_ROWS), jnp.float32),
    )(w, flat)


def _sc_topk_body(scores_hbm, idx_hbm, val_hbm, buf, idxb, valb, sem):
    wid = lax.axis_index("c") * NS + lax.axis_index("s")
    base = wid * RPW
    pltpu.sync_copy(scores_hbm.at[:, pl.ds(base, RPW)], buf)

    lane = lax.iota(jnp.int32, L)

    def group_body(g, carry):
        off = g * L
        s = [jnp.full((L,), -1.0, jnp.float32) for _ in range(TOPK)]
        ix = [jnp.zeros((L,), jnp.int32) for _ in range(TOPK)]
        for e in range(E):
            v = buf[e, pl.ds(off, L)]
            c = v > s[TOPK - 1]
            s[TOPK - 1] = jnp.maximum(s[TOPK - 1], v)
            ix[TOPK - 1] = jnp.where(c, jnp.full((L,), e, jnp.int32), ix[TOPK - 1])
            for j in range(TOPK - 1, 0, -1):
                cj = s[j] > s[j - 1]
                hi = jnp.maximum(s[j - 1], s[j])
                lo = jnp.minimum(s[j - 1], s[j])
                s[j - 1], s[j] = hi, lo
                ihi = jnp.where(cj, ix[j], ix[j - 1])
                ilo = jnp.where(cj, ix[j - 1], ix[j])
                ix[j - 1], ix[j] = ihi, ilo
        for j in range(TOPK):
            valb[j, pl.ds(off, L)] = s[j]
            idxb[j, pl.ds(off, L)] = ix[j]
        return carry

    lax.fori_loop(0, RPW // L, group_body, 0)

    pltpu.sync_copy(idxb, idx_hbm.at[:, pl.ds(base, RPW)])
    pltpu.sync_copy(valb, val_hbm.at[:, pl.ds(base, RPW)])


@functools.partial(
    pl.kernel,
    mesh=plsc.VectorSubcoreMesh(core_axis_name="c", subcore_axis_name="s"),
    out_type=[
        jax.ShapeDtypeStruct((TOPK, CH_ROWS), jnp.int32),
        jax.ShapeDtypeStruct((TOPK, CH_ROWS), jnp.float32),
    ],
    scratch_types=[
        pltpu.VMEM((E, RPW), jnp.float32),
        pltpu.VMEM((TOPK, RPW), jnp.int32),
        pltpu.VMEM((TOPK, RPW), jnp.float32),
        pltpu.SemaphoreType.DMA,
    ],
)
def _sc_topk(scores_hbm, idx_hbm, val_hbm, buf, idxb, valb, sem):
    _sc_topk_body(scores_hbm, idx_hbm, val_hbm, buf, idxb, valb, sem)


def kernel(hidden_states, W):
    flat = hidden_states.reshape(-1, H)
    idx_parts, val_parts = [], []
    for c in range(NCHUNK):
        scores_t = _tc_scores(flat, W, c)
        topk_idx_t, topk_val_t = _sc_topk(scores_t)
        idx_parts.append(topk_idx_t)
        val_parts.append(topk_val_t)
    topk_idx = jnp.concatenate(idx_parts, axis=1).T
    topk_val = jnp.concatenate(val_parts, axis=1).T
    return (topk_idx, topk_val)
